# Initial kernel scaffold; baseline (speedup 1.0000x reference)
import functools

import jax
import jax.numpy as jnp
from jax import lax
from jax.experimental import pallas as pl
from jax.experimental.pallas import tpu as pltpu
from jax.experimental.pallas import tpu_sc as plsc

N = 10000
E = 320000
D = 128
H = 256
L = 3

NC = 2
NS = 16
NWORK = NC * NS
CHUNK = 256
KSUB = CHUNK // 128
NCHUNKS = E // CHUNK
ITERS = (NCHUNKS + NWORK - 1) // NWORK
RPS = N // NS


def _sc_body(do_gather, e_hbm, src_hbm, dst_hbm, h_hbm, out_hbm,
             src_v, dst_v, e_v, rows_v, agg_sh, sem):
    c = lax.axis_index("c")
    s = lax.axis_index("s")
    wid = s * NC + c

    def _zrow(r, carry):
        for j in range(8):
            rows_v[r, pl.ds(j * 16, 16)] = jnp.zeros((16,), jnp.float32)
        return carry
    lax.fori_loop(0, CHUNK, _zrow, 0)
    for off, n in ((0, 256), (256, 256), (512, 113)):
        pltpu.sync_copy(rows_v.at[pl.ds(0, n)],
                        agg_sh.at[pl.ds(s * RPS + off, n)])
    plsc.subcore_barrier()

    def _chunk(i, carry):
        cid = wid + i * NWORK

        @pl.when(cid < NCHUNKS)
        def _():
            base = cid * CHUNK
            pltpu.sync_copy(src_hbm.at[pl.ds(cid * KSUB, KSUB)], src_v)
            pltpu.sync_copy(dst_hbm.at[pl.ds(cid * KSUB, KSUB)], dst_v)
            pltpu.sync_copy(e_hbm.at[pl.ds(base, CHUNK)], e_v)
            if do_gather:
                cps = [pltpu.async_copy(h_hbm.at[src_v.at[k]],
                                        rows_v.at[pl.ds(k * 128, 128)], sem)
                       for k in range(KSUB)]
                for cp in cps:
                    cp.wait()

            def _row(r, carry2):
                for j in range(8):
                    sl = pl.ds(j * 16, 16)
                    m = e_v[r, sl]
                    if do_gather:
                        m = m + rows_v[r, sl]
                    e_v[r, sl] = jnp.maximum(m, 0.0)
                return carry2
            lax.fori_loop(0, CHUNK, _row, 0)

            for k in range(KSUB):
                pltpu.sync_copy(e_v.at[pl.ds(k * 128, 128)],
                                agg_sh.at[dst_v.at[k]], add=True)
        return carry
    lax.fori_loop(0, ITERS, _chunk, 0)
    plsc.subcore_barrier()

    for off, n in ((0, 256), (256, 256), (512, 113)):
        pltpu.sync_copy(agg_sh.at[pl.ds(s * RPS + off, n)],
                        out_hbm.at[c, pl.ds(s * RPS + off, n)])


def _make_sc(do_gather):
    mesh = plsc.VectorSubcoreMesh(core_axis_name="c", subcore_axis_name="s")
    return pl.kernel(
        functools.partial(_sc_body, do_gather),
        out_type=jax.ShapeDtypeStruct((NC, N, D), jnp.float32),
        mesh=mesh,
        scratch_types=[
            pltpu.VMEM((KSUB, 128), jnp.int32),
            pltpu.VMEM((KSUB, 128), jnp.int32),
            pltpu.VMEM((CHUNK, D), jnp.float32),
            pltpu.VMEM((CHUNK, D), jnp.float32),
            pltpu.VMEM_SHARED((N, D), jnp.float32),
            pltpu.SemaphoreType.DMA,
        ],
    )


def _enc_body(ea_ref, we_ref, be_ref, out_ref):
    out_ref[0] = (jnp.dot(ea_ref[...], we_ref[0],
                          preferred_element_type=jnp.float32) + be_ref[0])


def _mlp_body(last, h_ref, agg_ref, ep_ref, w1_ref, b1_ref, w2_ref, b2_ref, out_ref):
    t = ep_ref[...] * h_ref[...] + agg_ref[0] + agg_ref[1]
    t = jnp.maximum(jnp.dot(t, w1_ref[...], preferred_element_type=jnp.float32)
                    + b1_ref[...], 0.0)
    o = jnp.dot(t, w2_ref[...], preferred_element_type=jnp.float32) + b2_ref[...]
    if not last:
        o = jnp.maximum(o, 0.0)
    out_ref[...] = o


_BE = 2000
_RB = 400


def kernel(x, edge_index, edge_attr, batch, node_table, We, be, eps,
           W1, b1, g1, bt1, m1, v1, W2, b2, go, bo, mo, vo):
    f32 = jnp.float32
    src2 = edge_index[0].reshape(E // 128, 128)
    dst2 = edge_index[1].reshape(E // 128, 128)
    h0row = node_table[0]

    s1 = g1 / jnp.sqrt(v1 + 1e-5)
    W1f = W1 * s1[:, None, :]
    b1f = b1 * s1 + (bt1 - m1 * s1)
    s2 = go / jnp.sqrt(vo + 1e-5)
    W2f = W2 * s2[:, None, :]
    b2f = b2 * s2 + (bo - mo * s2)

    ea_pad = jnp.pad(edge_attr, ((0, 0), (0, 1)))
    Wep = jnp.pad(We, ((0, 0), (0, 1), (0, 0)))
    bee = be.at[0].add(h0row).reshape(L, 1, D)

    e_all = pl.pallas_call(
        _enc_body,
        grid=(L, E // _BE),
        in_specs=[
            pl.BlockSpec((_BE, 8), lambda l, i: (i, 0)),
            pl.BlockSpec((1, 8, D), lambda l, i: (l, 0, 0)),
            pl.BlockSpec((1, 1, D), lambda l, i: (l, 0, 0)),
        ],
        out_specs=pl.BlockSpec((1, _BE, D), lambda l, i: (l, i, 0)),
        out_shape=jax.ShapeDtypeStruct((L, E, D), f32),
    )(ea_pad, Wep, bee)

    sc_first = _make_sc(False)
    sc_rest = _make_sc(True)

    h = jnp.broadcast_to(node_table[0:1], (N, D))
    for l in range(L):
        if l == 0:
            agg2 = sc_first(e_all[l], src2, dst2, h)
        else:
            agg2 = sc_rest(e_all[l], src2, dst2, h)
        epv = jnp.full((1, D), 1.0 + eps[l], f32)
        h = pl.pallas_call(
            functools.partial(_mlp_body, l == L - 1),
            grid=(N // _RB,),
            in_specs=[
                pl.BlockSpec((_RB, D), lambda i: (i, 0)),
                pl.BlockSpec((NC, _RB, D), lambda i: (0, i, 0)),
                pl.BlockSpec((1, D), lambda i: (0, 0)),
                pl.BlockSpec((D, H), lambda i: (0, 0)),
                pl.BlockSpec((1, H), lambda i: (0, 0)),
                pl.BlockSpec((H, D), lambda i: (0, 0)),
                pl.BlockSpec((1, D), lambda i: (0, 0)),
            ],
            out_specs=pl.BlockSpec((_RB, D), lambda i: (i, 0)),
            out_shape=jax.ShapeDtypeStruct((N, D), f32),
        )(h, agg2, epv, W1f[l], b1f[l].reshape(1, H), W2f[l], b2f[l].reshape(1, D))
    return h


# speedup vs baseline: 2.2276x; 2.2276x over previous
import functools

import jax
import jax.numpy as jnp
from jax import lax
from jax.experimental import pallas as pl
from jax.experimental.pallas import tpu as pltpu
from jax.experimental.pallas import tpu_sc as plsc

N = 10000
E = 320000
D = 128
H = 256
L = 3

NC = 2
NS = 16
NWORK = NC * NS
CHUNK = 128
KSUB = CHUNK // 128
NCHUNKS = E // CHUNK
ITERS = (NCHUNKS + NWORK - 1) // NWORK
RPS = 624


def _sc_body(do_gather, e_hbm, src_hbm, dst_hbm, h_hbm, out_hbm,
             src_v, dst_v, e_v, rows_v, agg_sh, sem):
    c = lax.axis_index("c")
    s = lax.axis_index("s")
    wid = s * NC + c

    def _zrow(r, carry):
        for j in range(8):
            rows_v[r, pl.ds(j * 16, 16)] = jnp.zeros((16,), jnp.float32)
        return carry
    lax.fori_loop(0, CHUNK, _zrow, 0)
    for off, n in ((0, 128), (128, 128), (256, 128), (384, 128), (512, 112)):
        pltpu.sync_copy(rows_v.at[pl.ds(0, n)],
                        agg_sh.at[pl.ds(s * RPS + off, n)])

    @pl.when(s == NS - 1)
    def _ztail():
        pltpu.sync_copy(rows_v.at[pl.ds(0, 16)], agg_sh.at[pl.ds(NS * RPS, 16)])
    plsc.subcore_barrier()

    def _chunk(i, carry):
        cid = wid + i * NWORK

        @pl.when(cid < NCHUNKS)
        def _():
            base = cid * CHUNK
            pltpu.sync_copy(src_hbm.at[pl.ds(cid * KSUB, KSUB)], src_v)
            pltpu.sync_copy(dst_hbm.at[pl.ds(cid * KSUB, KSUB)], dst_v)
            pltpu.sync_copy(e_hbm.at[pl.ds(base, CHUNK)], e_v)
            if do_gather:
                cps = [pltpu.async_copy(h_hbm.at[src_v.at[k]],
                                        rows_v.at[pl.ds(k * 128, 128)], sem)
                       for k in range(KSUB)]
                for cp in cps:
                    cp.wait()

            def _row(r, carry2):
                for j in range(8):
                    sl = pl.ds(j * 16, 16)
                    m = e_v[r, sl]
                    if do_gather:
                        m = m + rows_v[r, sl]
                    e_v[r, sl] = jnp.maximum(m, 0.0)
                return carry2
            lax.fori_loop(0, CHUNK, _row, 0)

            for k in range(KSUB):
                pltpu.sync_copy(e_v.at[pl.ds(k * 128, 128)],
                                agg_sh.at[dst_v.at[k]], add=True)
        return carry
    lax.fori_loop(0, ITERS, _chunk, 0)
    plsc.subcore_barrier()

    for off, n in ((0, 128), (128, 128), (256, 128), (384, 128), (512, 112)):
        pltpu.sync_copy(agg_sh.at[pl.ds(s * RPS + off, n)],
                        out_hbm.at[c, pl.ds(s * RPS + off, n)])

    @pl.when(s == NS - 1)
    def _wtail():
        pltpu.sync_copy(agg_sh.at[pl.ds(NS * RPS, 16)],
                        out_hbm.at[c, pl.ds(NS * RPS, 16)])


def _make_sc(do_gather):
    mesh = plsc.VectorSubcoreMesh(core_axis_name="c", subcore_axis_name="s")
    return pl.kernel(
        functools.partial(_sc_body, do_gather),
        out_type=jax.ShapeDtypeStruct((NC, N, D), jnp.float32),
        mesh=mesh,
        scratch_types=[
            pltpu.VMEM((KSUB, 128), jnp.int32),
            pltpu.VMEM((KSUB, 128), jnp.int32),
            pltpu.VMEM((CHUNK, D), jnp.float32),
            pltpu.VMEM((CHUNK, D), jnp.float32),
            pltpu.VMEM_SHARED((N, D), jnp.float32),
            pltpu.SemaphoreType.DMA,
        ],
    )


def _enc_body(ea_ref, we_ref, be_ref, out_ref):
    out_ref[0] = (jnp.dot(ea_ref[...], we_ref[0],
                          preferred_element_type=jnp.float32) + be_ref[0])


def _mlp_body(last, h_ref, agg_ref, ep_ref, w1_ref, b1_ref, w2_ref, b2_ref, out_ref):
    t = ep_ref[...] * h_ref[...] + agg_ref[0] + agg_ref[1]
    t = jnp.maximum(jnp.dot(t, w1_ref[...], preferred_element_type=jnp.float32)
                    + b1_ref[...], 0.0)
    o = jnp.dot(t, w2_ref[...], preferred_element_type=jnp.float32) + b2_ref[...]
    if not last:
        o = jnp.maximum(o, 0.0)
    out_ref[...] = o


_BE = 2000
_RB = 400


def kernel(x, edge_index, edge_attr, batch, node_table, We, be, eps,
           W1, b1, g1, bt1, m1, v1, W2, b2, go, bo, mo, vo):
    f32 = jnp.float32
    src2 = edge_index[0].reshape(E // 128, 128)
    dst2 = edge_index[1].reshape(E // 128, 128)
    h0row = node_table[0]

    s1 = g1 / jnp.sqrt(v1 + 1e-5)
    W1f = W1 * s1[:, None, :]
    b1f = b1 * s1 + (bt1 - m1 * s1)
    s2 = go / jnp.sqrt(vo + 1e-5)
    W2f = W2 * s2[:, None, :]
    b2f = b2 * s2 + (bo - mo * s2)

    ea_pad = jnp.pad(edge_attr, ((0, 0), (0, 1)))
    Wep = jnp.pad(We, ((0, 0), (0, 1), (0, 0)))
    bee = be.at[0].add(h0row).reshape(L, 1, D)

    e_all = pl.pallas_call(
        _enc_body,
        grid=(L, E // _BE),
        in_specs=[
            pl.BlockSpec((_BE, 8), lambda l, i: (i, 0)),
            pl.BlockSpec((1, 8, D), lambda l, i: (l, 0, 0)),
            pl.BlockSpec((1, 1, D), lambda l, i: (l, 0, 0)),
        ],
        out_specs=pl.BlockSpec((1, _BE, D), lambda l, i: (l, i, 0)),
        out_shape=jax.ShapeDtypeStruct((L, E, D), f32),
    )(ea_pad, Wep, bee)

    sc_first = _make_sc(False)
    sc_rest = _make_sc(True)

    h = jnp.broadcast_to(node_table[0:1], (N, D))
    for l in range(L):
        if l == 0:
            agg2 = sc_first(e_all[l], src2, dst2, h)
        else:
            agg2 = sc_rest(e_all[l], src2, dst2, h)
        epv = jnp.full((1, D), 1.0 + eps[l], f32)
        h = pl.pallas_call(
            functools.partial(_mlp_body, l == L - 1),
            grid=(N // _RB,),
            in_specs=[
                pl.BlockSpec((_RB, D), lambda i: (i, 0)),
                pl.BlockSpec((NC, _RB, D), lambda i: (0, i, 0)),
                pl.BlockSpec((1, D), lambda i: (0, 0)),
                pl.BlockSpec((D, H), lambda i: (0, 0)),
                pl.BlockSpec((1, H), lambda i: (0, 0)),
                pl.BlockSpec((H, D), lambda i: (0, 0)),
                pl.BlockSpec((1, D), lambda i: (0, 0)),
            ],
            out_specs=pl.BlockSpec((_RB, D), lambda i: (i, 0)),
            out_shape=jax.ShapeDtypeStruct((N, D), f32),
        )(h, agg2, epv, W1f[l], b1f[l].reshape(1, H), W2f[l], b2f[l].reshape(1, D))
    return h


# speedup vs baseline: 3.0250x; 1.3580x over previous
import functools

import jax
import jax.numpy as jnp
from jax import lax
from jax.experimental import pallas as pl
from jax.experimental.pallas import tpu as pltpu
from jax.experimental.pallas import tpu_sc as plsc

N = 10000
E = 320000
D = 128
H = 256
L = 3

NC = 2
NS = 16
NWORK = NC * NS
CHUNK = 128
KSUB = CHUNK // 128
NCHUNKS = E // CHUNK
ITERS = (NCHUNKS + NWORK - 1) // NWORK
RPS = 624


def _sc_body(do_gather, e_hbm, src_hbm, dst_hbm, h_hbm, out_hbm,
             src_v, dst_v, e_v, rows_v, agg_sh, sem):
    c = lax.axis_index("c")
    s = lax.axis_index("s")
    wid = s * NC + c

    def _zrow(r, carry):
        for j in range(8):
            rows_v[r, pl.ds(j * 16, 16)] = jnp.zeros((16,), jnp.float32)
        return carry
    lax.fori_loop(0, CHUNK, _zrow, 0)
    for off, n in ((0, 128), (128, 128), (256, 128), (384, 128), (512, 112)):
        pltpu.sync_copy(rows_v.at[pl.ds(0, n)],
                        agg_sh.at[pl.ds(s * RPS + off, n)])

    @pl.when(s == NS - 1)
    def _ztail():
        pltpu.sync_copy(rows_v.at[pl.ds(0, 16)], agg_sh.at[pl.ds(NS * RPS, 16)])
    plsc.subcore_barrier()

    def _chunk(i, carry):
        cid = wid + i * NWORK

        @pl.when(cid < NCHUNKS)
        def _():
            base = cid * CHUNK
            pltpu.sync_copy(src_hbm.at[pl.ds(cid * KSUB, KSUB)], src_v)
            pltpu.sync_copy(dst_hbm.at[pl.ds(cid * KSUB, KSUB)], dst_v)
            pltpu.sync_copy(e_hbm.at[pl.ds(base, CHUNK)], e_v)
            if do_gather:
                cps = [pltpu.async_copy(h_hbm.at[src_v.at[k]],
                                        rows_v.at[pl.ds(k * 128, 128)], sem)
                       for k in range(KSUB)]
                for cp in cps:
                    cp.wait()

            def _row(r, carry2):
                for j in range(8):
                    sl = pl.ds(j * 16, 16)
                    m = e_v[r, sl]
                    if do_gather:
                        m = m + rows_v[r, sl]
                    e_v[r, sl] = jnp.maximum(m, 0.0)
                return carry2
            lax.fori_loop(0, CHUNK, _row, 0)

            for k in range(KSUB):
                pltpu.sync_copy(e_v.at[pl.ds(k * 128, 128)],
                                agg_sh.at[dst_v.at[k]], add=True)
        return carry
    lax.fori_loop(0, ITERS, _chunk, 0)
    plsc.subcore_barrier()

    for off, n in ((0, 128), (128, 128), (256, 128), (384, 128), (512, 112)):
        pltpu.sync_copy(agg_sh.at[pl.ds(s * RPS + off, n)],
                        out_hbm.at[c, pl.ds(s * RPS + off, n)])

    @pl.when(s == NS - 1)
    def _wtail():
        pltpu.sync_copy(agg_sh.at[pl.ds(NS * RPS, 16)],
                        out_hbm.at[c, pl.ds(NS * RPS, 16)])


def _make_sc(do_gather):
    mesh = plsc.VectorSubcoreMesh(core_axis_name="c", subcore_axis_name="s")
    return pl.kernel(
        functools.partial(_sc_body, do_gather),
        out_type=jax.ShapeDtypeStruct((NC, N, D), jnp.float32),
        mesh=mesh,
        scratch_types=[
            pltpu.VMEM((KSUB, 128), jnp.int32),
            pltpu.VMEM((KSUB, 128), jnp.int32),
            pltpu.VMEM((CHUNK, D), jnp.float32),
            pltpu.VMEM((CHUNK, D), jnp.float32),
            pltpu.VMEM_SHARED((N, D), jnp.float32),
            pltpu.SemaphoreType.DMA,
        ],
    )


def _enc_body(ea_ref, we_ref, be_ref, *out_refs):
    ea = ea_ref[...]
    for l, out_ref in enumerate(out_refs):
        out_ref[...] = (jnp.dot(ea, we_ref[l],
                                preferred_element_type=jnp.float32) + be_ref[l])


def _mlp_body(last, h_ref, agg_ref, ep_ref, w1_ref, b1_ref, w2_ref, b2_ref, out_ref):
    t = ep_ref[...] * h_ref[...] + agg_ref[0] + agg_ref[1]
    t = jnp.maximum(jnp.dot(t, w1_ref[...], preferred_element_type=jnp.float32)
                    + b1_ref[...], 0.0)
    o = jnp.dot(t, w2_ref[...], preferred_element_type=jnp.float32) + b2_ref[...]
    if not last:
        o = jnp.maximum(o, 0.0)
    out_ref[...] = o


_BE = 2000
_RB = 400


def kernel(x, edge_index, edge_attr, batch, node_table, We, be, eps,
           W1, b1, g1, bt1, m1, v1, W2, b2, go, bo, mo, vo):
    f32 = jnp.float32
    src2 = edge_index[0].reshape(E // 128, 128)
    dst2 = edge_index[1].reshape(E // 128, 128)
    h0row = node_table[0]

    s1 = g1 / jnp.sqrt(v1 + 1e-5)
    W1f = W1 * s1[:, None, :]
    b1f = b1 * s1 + (bt1 - m1 * s1)
    s2 = go / jnp.sqrt(vo + 1e-5)
    W2f = W2 * s2[:, None, :]
    b2f = b2 * s2 + (bo - mo * s2)

    ea_pad = jnp.pad(edge_attr, ((0, 0), (0, 1)))
    Wep = jnp.pad(We, ((0, 0), (0, 1), (0, 0)))
    bee = be.at[0].add(h0row).reshape(L, 1, D)

    e_all = pl.pallas_call(
        _enc_body,
        grid=(E // _BE,),
        in_specs=[
            pl.BlockSpec((_BE, 8), lambda i: (i, 0)),
            pl.BlockSpec((L, 8, D), lambda i: (0, 0, 0)),
            pl.BlockSpec((L, 1, D), lambda i: (0, 0, 0)),
        ],
        out_specs=[pl.BlockSpec((_BE, D), lambda i: (i, 0)) for _ in range(L)],
        out_shape=[jax.ShapeDtypeStruct((E, D), f32) for _ in range(L)],
    )(ea_pad, Wep, bee)

    sc_first = _make_sc(False)
    sc_rest = _make_sc(True)

    h = jnp.broadcast_to(node_table[0:1], (N, D))
    for l in range(L):
        if l == 0:
            agg2 = sc_first(e_all[l], src2, dst2, h)
        else:
            agg2 = sc_rest(e_all[l], src2, dst2, h)
        epv = jnp.full((1, D), 1.0 + eps[l], f32)
        h = pl.pallas_call(
            functools.partial(_mlp_body, l == L - 1),
            grid=(N // _RB,),
            in_specs=[
                pl.BlockSpec((_RB, D), lambda i: (i, 0)),
                pl.BlockSpec((NC, _RB, D), lambda i: (0, i, 0)),
                pl.BlockSpec((1, D), lambda i: (0, 0)),
                pl.BlockSpec((D, H), lambda i: (0, 0)),
                pl.BlockSpec((1, H), lambda i: (0, 0)),
                pl.BlockSpec((H, D), lambda i: (0, 0)),
                pl.BlockSpec((1, D), lambda i: (0, 0)),
            ],
            out_specs=pl.BlockSpec((_RB, D), lambda i: (i, 0)),
            out_shape=jax.ShapeDtypeStruct((N, D), f32),
        )(h, agg2, epv, W1f[l], b1f[l].reshape(1, H), W2f[l], b2f[l].reshape(1, D))
    return h
